# baseline (speedup 1.0000x reference)
import functools
import math

import jax
import jax.numpy as jnp
from jax import lax
from jax.experimental import pallas as pl
from jax.experimental.pallas import tpu as pltpu
from jax.experimental.pallas import tpu_sc as plsc

_K = 10
_A = 1.0
_B = 1.0
_ALPHA = 0.5

_NS = 16
_NC = 2
_CH = 128


def _adjust_ab(a, b):
    if a + b <= -1.0:
        gap = -a - b - 1.0 + 0.0001
        a = a + gap / 2
        b = b + gap / 2
    return a, b


def _jacobi_abc(n):
    a, b = _adjust_ab(_A, _B)
    nab = 2 * n + a + b
    denom = 2 * n * (nab - n) * (nab - 2)
    an = nab * (nab - 1) * (nab - 2) / denom
    bn = (nab - 1) * (a * a - b * b) / denom
    cn = -2 * (n + a - 1) * (n + b - 1) * nab / denom
    return an, bn, cn


def _norm_weights():
    a, b = _adjust_ab(_A, _B)
    ws = []
    for i in range(_K + 1):
        term1 = (2.0 ** (a + b + 1)) / (2 * i + a + b + 1)
        term2 = math.exp(math.lgamma(i + a + 1) - math.lgamma(i + a + b + 1))
        term3 = math.exp(math.lgamma(i + b + 1) - math.lgamma(i + 1))
        ws.append(math.sqrt(term1 * term2 * term3))
    return ws


def _sc_mesh():
    return plsc.VectorSubcoreMesh(
        core_axis_name="c", subcore_axis_name="s",
        num_cores=_NC, num_subcores=_NS)


def _make_deg_kernel(n, nch):
    nt = n + 8
    npt = n // _NS
    ncs = 125
    nck = npt // ncs
    ect = nch // (_NC * _NS)

    @functools.partial(
        pl.kernel,
        out_type=jax.ShapeDtypeStruct((_NC, n, 16), jnp.float32),
        mesh=_sc_mesh(),
        compiler_params=pltpu.CompilerParams(use_tc_tiling_on_sc=False),
        scratch_types=[
            pltpu.VMEM_SHARED((nt, 16), jnp.float32),
            pltpu.VMEM((_CH, 16), jnp.float32),
            pltpu.VMEM((_CH,), jnp.int32),
        ],
    )
    def degk(col_hbm, t2_hbm, t2_sp, onesb, colv):
        c = lax.axis_index("c")
        s = lax.axis_index("s")
        node_base = s * npt
        zero16 = jnp.zeros((16,), jnp.float32)
        one16 = jnp.ones((16,), jnp.float32)

        def zrow(r, carry):
            onesb[r, pl.ds(0, 16)] = zero16
            return carry
        lax.fori_loop(0, _CH, zrow, 0)

        def zcp(ch, carry):
            pltpu.sync_copy(onesb.at[pl.ds(0, ncs)],
                            t2_sp.at[pl.ds(node_base + ch * ncs, ncs)])
            return carry
        lax.fori_loop(0, nck, zcp, 0)

        @pl.when(s == _NS - 1)
        def _():
            pltpu.sync_copy(onesb.at[pl.ds(0, 8)], t2_sp.at[pl.ds(n, 8)])

        def orow(r, carry):
            onesb[r, pl.ds(0, 16)] = one16
            return carry
        lax.fori_loop(0, _CH, orow, 0)

        plsc.subcore_barrier()

        def echunk(jj, carry):
            j = c * (nch // _NC) + s * ect + jj
            pltpu.sync_copy(col_hbm.at[j], colv)
            pltpu.sync_copy(onesb, t2_sp.at[colv], add=True)
            return carry
        lax.fori_loop(0, ect, echunk, 0)

        plsc.subcore_barrier()

        def wout(ch, carry):
            sl = pl.ds(node_base + ch * ncs, ncs)
            pltpu.sync_copy(t2_sp.at[sl], t2_hbm.at[c].at[sl])
            return carry
        lax.fori_loop(0, nck, wout, 0)

    return degk


def _prep_body(t2_ref, x_ref, dinvx_ref, g0_ref, xs_ref):
    t2 = t2_ref[0] + t2_ref[1]
    deg = jnp.sum(t2, axis=1) * (1.0 / 16.0)
    dinv = jnp.where(deg > 0.0, lax.rsqrt(jnp.where(deg > 0.0, deg, 1.0)), 0.0)
    dcol = dinv[:, None]
    blk, d2 = dinvx_ref.shape
    dinvx_ref[...] = jnp.broadcast_to(dcol, (blk, d2))
    x = x_ref[...]
    xs_ref[0] = x[:, :d2]
    xs_ref[1] = x[:, d2:]
    g0_ref[0] = dcol * x[:, :d2]
    g0_ref[1] = dcol * x[:, d2:]


def _make_step_kernel(n, nch):
    nt = n + 8
    npt = n // _NS
    ncs = 125
    nck = npt // ncs
    ect = nch // _NS
    fdim = 64

    vrow = functools.partial(pltpu.VMEM, (ncs, fdim))
    nbuf = 2

    @functools.partial(
        pl.kernel,
        out_type=[jax.ShapeDtypeStruct((_NC * n, fdim), jnp.float32)] * 3,
        mesh=_sc_mesh(),
        compiler_params=pltpu.CompilerParams(use_tc_tiling_on_sc=False),
        scratch_types=(
            [pltpu.VMEM_SHARED((nt, fdim), jnp.float32)]
            + [pltpu.VMEM((_CH, fdim), jnp.float32)] * 4
            + [vrow(jnp.float32)] * 6
            + [pltpu.VMEM((_CH,), jnp.int32)] * 8
            + [pltpu.VMEM((80,), jnp.float32)]
            + [pltpu.SemaphoreType.DMA] * 12
        ),
    )
    def step(pa, pb, g, acc, dinvx, row2, col, coefs,
             pnext, gnext, accout,
             t_sp, ev0, ev1, ev2, ev3,
             tv, pav, pbv, accv, dxv, pnv,
             rowv0, rowv1, rowv2, rowv3,
             colv0, colv1, colv2, colv3, cfv,
             semg0, semg1, semg2, semg3,
             sems0, sems1, sems2, sems3,
             semi0, semi1, semi2, semi3):
        c = lax.axis_index("c")
        s = lax.axis_index("s")
        node_base = s * npt
        hbm_base = c * n + node_base
        zero16 = jnp.zeros((16,), jnp.float32)
        evs = [ev0, ev1, ev2, ev3]
        rowvs = [rowv0, rowv1, rowv2, rowv3]
        colvs = [colv0, colv1, colv2, colv3]
        semgs = [semg0, semg1, semg2, semg3]
        semss = [sems0, sems1, sems2, sems3]
        semis = [semi0, semi1, semi2, semi3]
        jt = s * ect

        pltpu.sync_copy(coefs, cfv)

        def fire_idx(i, e):
            pltpu.sync_copy(row2.at[c].at[jt + e], rowvs[i])
            pltpu.sync_copy(col.at[jt + e], colvs[i])

        def wait_idx(i, e):
            pass

        def fire_gather(i):
            pltpu.async_copy(g.at[rowvs[i]], evs[i], semgs[i])

        def wait_gather(i):
            pltpu.make_async_copy(g.at[rowvs[i]], evs[i], semgs[i]).wait()

        def fire_scatter(i):
            pltpu.async_copy(evs[i], t_sp.at[colvs[i]], semss[i], add=True)

        def wait_scatter(i):
            pltpu.make_async_copy(evs[i], t_sp.at[colvs[i]], semss[i]).wait()

        for i in range(nbuf):
            fire_idx(i, i)
        for i in range(nbuf):
            wait_idx(i, i)
            fire_gather(i)

        def zrow(r, carry):
            for j2 in range(fdim // 16):
                pnv[r, pl.ds(j2 * 16, 16)] = zero16
            return carry
        lax.fori_loop(0, ncs, zrow, 0)

        def zcp(ch, carry):
            pltpu.sync_copy(pnv, t_sp.at[pl.ds(node_base + ch * ncs, ncs)])
            return carry
        lax.fori_loop(0, nck, zcp, 0)

        @pl.when(s == _NS - 1)
        def _():
            pltpu.sync_copy(pnv.at[pl.ds(0, 8)], t_sp.at[pl.ds(n, 8)])

        plsc.subcore_barrier()

        def egroup(gi, carry):
            e0 = nbuf * gi
            for i in range(nbuf):
                wait_gather(i)
                pltpu.async_copy(evs[i], t_sp.at[colvs[i]],
                                 semss[i], add=True).wait()
            for i in range(nbuf):
                fire_idx(i, e0 + nbuf + i)
                fire_gather(i)
            return carry
        lax.fori_loop(0, ect // nbuf - 1, egroup, 0)

        for i in range(nbuf):
            wait_gather(i)
            pltpu.async_copy(evs[i], t_sp.at[colvs[i]],
                             semss[i], add=True).wait()

        plsc.subcore_barrier()

        bv = cfv[pl.ds(0, 16)]
        cv = cfv[pl.ds(16, 16)]
        av = cfv[pl.ds(32, 16)]
        ov = cfv[pl.ds(48, 16)]
        dv = cfv[pl.ds(64, 16)]

        def nchunk(ch, carry):
            hb = pl.ds(hbm_base + ch * ncs, ncs)
            nb = pl.ds(node_base + ch * ncs, ncs)
            pltpu.sync_copy(t_sp.at[nb], tv)
            pltpu.sync_copy(pa.at[hb], pav)
            pltpu.sync_copy(pb.at[hb], pbv)
            pltpu.sync_copy(acc.at[hb], accv)
            pltpu.sync_copy(dinvx.at[nb], dxv)

            def rrow(r, carry2):
                for j2 in range(fdim // 16):
                    sl = pl.ds(j2 * 16, 16)
                    pb_ = pbv[r, sl]
                    pn = (bv * pb_ + cv * pav[r, sl]
                          + av * (dxv[r, sl] * tv[r, sl]))
                    pnv[r, sl] = pn
                    ev0[r, sl] = dxv[r, sl] * pn
                    accv[r, sl] = accv[r, sl] + ov * pn + dv * pb_
                return carry2
            lax.fori_loop(0, ncs, rrow, 0)

            pltpu.sync_copy(pnv, pnext.at[hb])
            pltpu.sync_copy(ev0.at[pl.ds(0, ncs)], gnext.at[hb])
            pltpu.sync_copy(accv, accout.at[hb])
            return carry
        lax.fori_loop(0, nck, nchunk, 0)

    return step


def kernel(x, edge_index, lap_coefs, mf_weights):
    n, d = x.shape
    e = edge_index.shape[1]
    row = edge_index[0]
    col = edge_index[1]

    nch = -(-e // _CH)
    nch = -(-nch // (_NC * _NS)) * (_NC * _NS)
    pad = nch * _CH - e
    row_p = jnp.concatenate([row, jnp.zeros((pad,), jnp.int32)])
    col_p = jnp.concatenate([col, jnp.full((pad,), n, jnp.int32)])
    row_r = row_p.reshape(nch, _CH)
    col_r = col_p.reshape(nch, _CH)
    row2 = jnp.stack([row_r, row_r + n])

    t2 = _make_deg_kernel(n, nch)(col_r)
    blk = 1000
    d2 = d // 2
    dinvx, g0, xs = pl.pallas_call(
        _prep_body,
        out_shape=[
            jax.ShapeDtypeStruct((n, d2), jnp.float32),
            jax.ShapeDtypeStruct((_NC, n, d2), jnp.float32),
            jax.ShapeDtypeStruct((_NC, n, d2), jnp.float32),
        ],
        grid=(n // blk,),
        in_specs=[
            pl.BlockSpec((_NC, blk, 16), lambda i: (0, i, 0)),
            pl.BlockSpec((blk, d), lambda i: (i, 0)),
        ],
        out_specs=[
            pl.BlockSpec((blk, d2), lambda i: (i, 0)),
            pl.BlockSpec((_NC, blk, d2), lambda i: (0, i, 0)),
            pl.BlockSpec((_NC, blk, d2), lambda i: (0, i, 0)),
        ],
    )(t2, x)
    g0 = g0.reshape(_NC * n, d2)
    xs = xs.reshape(_NC * n, d2)

    lap = jnp.cumprod(_ALPHA * jnp.tanh(lap_coefs))
    nw = _norm_weights()
    mfw = mf_weights[0, :, 0]
    cout = [mfw[0] / nw[0]]
    for i in range(1, _K + 1):
        cout.append(lap[i - 1] / nw[i] * mfw[i])

    a, b = _adjust_ab(_A, _B)
    c0 = (a - b) / 2.0
    c1 = (a + b + 2.0) / 2.0

    def coef_vec(v):
        return jnp.full((16,), v, jnp.float32)

    step = _make_step_kernel(n, nch)
    pa, pb, g = xs, xs, g0
    acc = jnp.zeros((_NC * n, d2), jnp.float32)
    for kc in range(_K):
        if kc == 0:
            bn, cn, an = c0, 0.0, c1
            dvc = cout[0]
        else:
            an, bn, cn = _jacobi_abc(kc + 1)
            dvc = 0.0
        coefs = jnp.concatenate([
            coef_vec(bn), coef_vec(cn), coef_vec(an),
            coef_vec(cout[kc + 1]), coef_vec(dvc),
        ])
        pnext, gnext, accout = step(pa, pb, g, acc,
                                    dinvx, row2, col_r, coefs)
        pa, pb, g, acc = pb, pnext, gnext, accout

    return jnp.concatenate([acc[:n], acc[n:]], axis=1)


# speedup vs baseline: 1.3343x; 1.3343x over previous
import functools
import math

import jax
import jax.numpy as jnp
from jax import lax
from jax.experimental import pallas as pl
from jax.experimental.pallas import tpu as pltpu
from jax.experimental.pallas import tpu_sc as plsc

_K = 10
_A = 1.0
_B = 1.0
_ALPHA = 0.5

_NS = 16
_NC = 2
_CH = 128


def _adjust_ab(a, b):
    if a + b <= -1.0:
        gap = -a - b - 1.0 + 0.0001
        a = a + gap / 2
        b = b + gap / 2
    return a, b


def _jacobi_abc(n):
    a, b = _adjust_ab(_A, _B)
    nab = 2 * n + a + b
    denom = 2 * n * (nab - n) * (nab - 2)
    an = nab * (nab - 1) * (nab - 2) / denom
    bn = (nab - 1) * (a * a - b * b) / denom
    cn = -2 * (n + a - 1) * (n + b - 1) * nab / denom
    return an, bn, cn


def _norm_weights():
    a, b = _adjust_ab(_A, _B)
    ws = []
    for i in range(_K + 1):
        term1 = (2.0 ** (a + b + 1)) / (2 * i + a + b + 1)
        term2 = math.exp(math.lgamma(i + a + 1) - math.lgamma(i + a + b + 1))
        term3 = math.exp(math.lgamma(i + b + 1) - math.lgamma(i + 1))
        ws.append(math.sqrt(term1 * term2 * term3))
    return ws


def _sc_mesh():
    return plsc.VectorSubcoreMesh(
        core_axis_name="c", subcore_axis_name="s",
        num_cores=_NC, num_subcores=_NS)


def _make_deg_kernel(n, nch):
    nt = n + 8
    npt = n // _NS
    ncs = 125
    nck = npt // ncs
    ect = nch // (_NC * _NS)

    @functools.partial(
        pl.kernel,
        out_type=jax.ShapeDtypeStruct((_NC, n, 16), jnp.float32),
        mesh=_sc_mesh(),
        compiler_params=pltpu.CompilerParams(use_tc_tiling_on_sc=False),
        scratch_types=[
            pltpu.VMEM_SHARED((nt, 16), jnp.float32),
            pltpu.VMEM((_CH, 16), jnp.float32),
            pltpu.VMEM((_CH,), jnp.int32),
        ],
    )
    def degk(col_hbm, t2_hbm, t2_sp, onesb, colv):
        c = lax.axis_index("c")
        s = lax.axis_index("s")
        node_base = s * npt
        zero16 = jnp.zeros((16,), jnp.float32)
        one16 = jnp.ones((16,), jnp.float32)

        def zrow(r, carry):
            onesb[r, pl.ds(0, 16)] = zero16
            return carry
        lax.fori_loop(0, _CH, zrow, 0)

        def zcp(ch, carry):
            pltpu.sync_copy(onesb.at[pl.ds(0, ncs)],
                            t2_sp.at[pl.ds(node_base + ch * ncs, ncs)])
            return carry
        lax.fori_loop(0, nck, zcp, 0)

        @pl.when(s == _NS - 1)
        def _():
            pltpu.sync_copy(onesb.at[pl.ds(0, 8)], t2_sp.at[pl.ds(n, 8)])

        def orow(r, carry):
            onesb[r, pl.ds(0, 16)] = one16
            return carry
        lax.fori_loop(0, _CH, orow, 0)

        plsc.subcore_barrier()

        def echunk(jj, carry):
            j = c * (nch // _NC) + s * ect + jj
            pltpu.sync_copy(col_hbm.at[j], colv)
            pltpu.sync_copy(onesb, t2_sp.at[colv], add=True)
            return carry
        lax.fori_loop(0, ect, echunk, 0)

        plsc.subcore_barrier()

        def wout(ch, carry):
            sl = pl.ds(node_base + ch * ncs, ncs)
            pltpu.sync_copy(t2_sp.at[sl], t2_hbm.at[c].at[sl])
            return carry
        lax.fori_loop(0, nck, wout, 0)

    return degk


def _prep_body(t2_ref, x_ref, dinvx_ref, g0_ref, xs_ref):
    t2 = t2_ref[0] + t2_ref[1]
    deg = jnp.sum(t2, axis=1) * (1.0 / 16.0)
    dinv = jnp.where(deg > 0.0, lax.rsqrt(jnp.where(deg > 0.0, deg, 1.0)), 0.0)
    dcol = dinv[:, None]
    blk, d2 = dinvx_ref.shape
    dinvx_ref[...] = jnp.broadcast_to(dcol, (blk, d2))
    x = x_ref[...]
    xs_ref[0] = x[:, :d2]
    xs_ref[1] = x[:, d2:]
    g0_ref[0] = dcol * x[:, :d2]
    g0_ref[1] = dcol * x[:, d2:]


def _make_step_kernel(n, nch):
    nt = n + 8
    npt = n // _NS
    ncs = 125
    nck = npt // ncs
    ect = nch // _NS
    fdim = 64

    vrow = functools.partial(pltpu.VMEM, (ncs, fdim))

    @functools.partial(
        pl.kernel,
        out_type=[jax.ShapeDtypeStruct((_NC * n, fdim), jnp.float32)] * 3,
        mesh=_sc_mesh(),
        compiler_params=pltpu.CompilerParams(use_tc_tiling_on_sc=False),
        scratch_types=[
            pltpu.VMEM_SHARED((nt, fdim), jnp.float32),
            pltpu.VMEM((_CH, fdim), jnp.float32),
            pltpu.VMEM((_CH, fdim), jnp.float32),
            vrow(jnp.float32), vrow(jnp.float32), vrow(jnp.float32),
            vrow(jnp.float32), vrow(jnp.float32), vrow(jnp.float32),
            pltpu.VMEM((ect, _CH), jnp.int32),
            pltpu.VMEM((_CH,), jnp.int32),
            pltpu.VMEM((_CH,), jnp.int32),
            pltpu.VMEM((_CH,), jnp.int32),
            pltpu.VMEM((_CH,), jnp.int32),
            pltpu.VMEM((80,), jnp.float32),
            pltpu.SemaphoreType.DMA,
            pltpu.SemaphoreType.DMA,
            pltpu.SemaphoreType.DMA,
            pltpu.SemaphoreType.DMA,
            pltpu.SemaphoreType.DMA,
            pltpu.SemaphoreType.DMA,
        ],
    )
    def step(pa, pb, g, acc, dinvx, row2, col, coefs,
             pnext, gnext, accout,
             t_sp, eva, evb, tv, pav, pbv, accv, dxv, pnv,
             colb, rowva, rowvb, colva, colvb, cfv,
             semga, semgb, semsa, semsb, semia, semib):
        c = lax.axis_index("c")
        s = lax.axis_index("s")
        node_base = s * npt
        hbm_base = c * n + node_base
        zero16 = jnp.zeros((16,), jnp.float32)

        pltpu.sync_copy(coefs, cfv)

        jt = s * ect
        pltpu.sync_copy(col.at[pl.ds(jt, ect)], colb)
        pltpu.sync_copy(row2.at[c].at[jt], rowva)
        pltpu.sync_copy(row2.at[c].at[jt + 1], rowvb)
        pltpu.async_copy(g.at[rowva], eva, semga)
        pltpu.async_copy(g.at[rowvb], evb, semgb)

        def zrow(r, carry):
            for j2 in range(fdim // 16):
                pnv[r, pl.ds(j2 * 16, 16)] = zero16
            return carry
        lax.fori_loop(0, ncs, zrow, 0)

        def zcp(ch, carry):
            pltpu.sync_copy(pnv, t_sp.at[pl.ds(node_base + ch * ncs, ncs)])
            return carry
        lax.fori_loop(0, nck, zcp, 0)

        @pl.when(s == _NS - 1)
        def _():
            pltpu.sync_copy(pnv.at[pl.ds(0, 8)], t_sp.at[pl.ds(n, 8)])

        plsc.subcore_barrier()

        npair = ect // 2

        def epair(p, carry):
            e = 2 * p

            pltpu.make_async_copy(g.at[rowva], eva, semga).wait()
            pltpu.async_copy(row2.at[c].at[jt + e + 2], rowva, semia)
            for k16 in range(_CH // 16):
                sl = pl.ds(k16 * 16, 16)
                colva[sl] = colb[e, sl]
            da = pltpu.async_copy(eva, t_sp.at[colva], semsa, add=True)

            pltpu.make_async_copy(g.at[rowvb], evb, semgb).wait()
            pltpu.async_copy(row2.at[c].at[jt + e + 3], rowvb, semib)
            for k16 in range(_CH // 16):
                sl = pl.ds(k16 * 16, 16)
                colvb[sl] = colb[e + 1, sl]
            db = pltpu.async_copy(evb, t_sp.at[colvb], semsb, add=True)

            da.wait()
            pltpu.make_async_copy(row2.at[c].at[jt + e + 2], rowva,
                                  semia).wait()
            pltpu.async_copy(g.at[rowva], eva, semga)
            db.wait()
            pltpu.make_async_copy(row2.at[c].at[jt + e + 3], rowvb,
                                  semib).wait()
            pltpu.async_copy(g.at[rowvb], evb, semgb)
            return carry
        lax.fori_loop(0, npair - 1, epair, 0)

        e = ect - 2
        pltpu.make_async_copy(g.at[rowva], eva, semga).wait()
        for k16 in range(_CH // 16):
            sl = pl.ds(k16 * 16, 16)
            colva[sl] = colb[e, sl]
        da = pltpu.async_copy(eva, t_sp.at[colva], semsa, add=True)
        pltpu.make_async_copy(g.at[rowvb], evb, semgb).wait()
        for k16 in range(_CH // 16):
            sl = pl.ds(k16 * 16, 16)
            colvb[sl] = colb[e + 1, sl]
        db = pltpu.async_copy(evb, t_sp.at[colvb], semsb, add=True)
        da.wait()
        db.wait()

        plsc.subcore_barrier()

        bv = cfv[pl.ds(0, 16)]
        cv = cfv[pl.ds(16, 16)]
        av = cfv[pl.ds(32, 16)]
        ov = cfv[pl.ds(48, 16)]
        dv = cfv[pl.ds(64, 16)]

        def nchunk(ch, carry):
            hb = pl.ds(hbm_base + ch * ncs, ncs)
            nb = pl.ds(node_base + ch * ncs, ncs)
            pltpu.sync_copy(t_sp.at[nb], tv)
            pltpu.sync_copy(pa.at[hb], pav)
            pltpu.sync_copy(pb.at[hb], pbv)
            pltpu.sync_copy(acc.at[hb], accv)
            pltpu.sync_copy(dinvx.at[nb], dxv)

            def rrow(r, carry2):
                for j2 in range(fdim // 16):
                    sl = pl.ds(j2 * 16, 16)
                    pb_ = pbv[r, sl]
                    pn = (bv * pb_ + cv * pav[r, sl]
                          + av * (dxv[r, sl] * tv[r, sl]))
                    pnv[r, sl] = pn
                    eva[r, sl] = dxv[r, sl] * pn
                    accv[r, sl] = accv[r, sl] + ov * pn + dv * pb_
                return carry2
            lax.fori_loop(0, ncs, rrow, 0)

            pltpu.sync_copy(pnv, pnext.at[hb])
            pltpu.sync_copy(eva.at[pl.ds(0, ncs)], gnext.at[hb])
            pltpu.sync_copy(accv, accout.at[hb])
            return carry
        lax.fori_loop(0, nck, nchunk, 0)

    return step


def kernel(x, edge_index, lap_coefs, mf_weights):
    n, d = x.shape
    e = edge_index.shape[1]
    row = edge_index[0]
    col = edge_index[1]

    nch = -(-e // _CH)
    nch = -(-nch // (_NC * _NS)) * (_NC * _NS)
    pad = nch * _CH - e
    row_p = jnp.concatenate([row, jnp.zeros((pad,), jnp.int32)])
    col_p = jnp.concatenate([col, jnp.full((pad,), n, jnp.int32)])
    row_r = row_p.reshape(nch, _CH)
    col_r = col_p.reshape(nch, _CH)
    row2 = jnp.stack([row_r, row_r + n])

    t2 = _make_deg_kernel(n, nch)(col_r)
    blk = 1000
    d2 = d // 2
    dinvx, g0, xs = pl.pallas_call(
        _prep_body,
        out_shape=[
            jax.ShapeDtypeStruct((n, d2), jnp.float32),
            jax.ShapeDtypeStruct((_NC, n, d2), jnp.float32),
            jax.ShapeDtypeStruct((_NC, n, d2), jnp.float32),
        ],
        grid=(n // blk,),
        in_specs=[
            pl.BlockSpec((_NC, blk, 16), lambda i: (0, i, 0)),
            pl.BlockSpec((blk, d), lambda i: (i, 0)),
        ],
        out_specs=[
            pl.BlockSpec((blk, d2), lambda i: (i, 0)),
            pl.BlockSpec((_NC, blk, d2), lambda i: (0, i, 0)),
            pl.BlockSpec((_NC, blk, d2), lambda i: (0, i, 0)),
        ],
    )(t2, x)
    g0 = g0.reshape(_NC * n, d2)
    xs = xs.reshape(_NC * n, d2)

    lap = jnp.cumprod(_ALPHA * jnp.tanh(lap_coefs))
    nw = _norm_weights()
    mfw = mf_weights[0, :, 0]
    cout = [mfw[0] / nw[0]]
    for i in range(1, _K + 1):
        cout.append(lap[i - 1] / nw[i] * mfw[i])

    a, b = _adjust_ab(_A, _B)
    c0 = (a - b) / 2.0
    c1 = (a + b + 2.0) / 2.0

    def coef_vec(v):
        return jnp.full((16,), v, jnp.float32)

    step = _make_step_kernel(n, nch)
    pa, pb, g = xs, xs, g0
    acc = jnp.zeros((_NC * n, d2), jnp.float32)
    for kc in range(_K):
        if kc == 0:
            bn, cn, an = c0, 0.0, c1
            dvc = cout[0]
        else:
            an, bn, cn = _jacobi_abc(kc + 1)
            dvc = 0.0
        coefs = jnp.concatenate([
            coef_vec(bn), coef_vec(cn), coef_vec(an),
            coef_vec(cout[kc + 1]), coef_vec(dvc),
        ])
        pnext, gnext, accout = step(pa, pb, g, acc,
                                    dinvx, row2, col_r, coefs)
        pa, pb, g, acc = pb, pnext, gnext, accout

    return jnp.concatenate([acc[:n], acc[n:]], axis=1)


# speedup vs baseline: 1.3343x; 1.0000x over previous
import functools
import math

import jax
import jax.numpy as jnp
from jax import lax
from jax.experimental import pallas as pl
from jax.experimental.pallas import tpu as pltpu
from jax.experimental.pallas import tpu_sc as plsc

_K = 10
_A = 1.0
_B = 1.0
_ALPHA = 0.5

_NS = 16
_NC = 2
_CH = 128


def _adjust_ab(a, b):
    if a + b <= -1.0:
        gap = -a - b - 1.0 + 0.0001
        a = a + gap / 2
        b = b + gap / 2
    return a, b


def _jacobi_abc(n):
    a, b = _adjust_ab(_A, _B)
    nab = 2 * n + a + b
    denom = 2 * n * (nab - n) * (nab - 2)
    an = nab * (nab - 1) * (nab - 2) / denom
    bn = (nab - 1) * (a * a - b * b) / denom
    cn = -2 * (n + a - 1) * (n + b - 1) * nab / denom
    return an, bn, cn


def _norm_weights():
    a, b = _adjust_ab(_A, _B)
    ws = []
    for i in range(_K + 1):
        term1 = (2.0 ** (a + b + 1)) / (2 * i + a + b + 1)
        term2 = math.exp(math.lgamma(i + a + 1) - math.lgamma(i + a + b + 1))
        term3 = math.exp(math.lgamma(i + b + 1) - math.lgamma(i + 1))
        ws.append(math.sqrt(term1 * term2 * term3))
    return ws


def _sc_mesh():
    return plsc.VectorSubcoreMesh(
        core_axis_name="c", subcore_axis_name="s",
        num_cores=_NC, num_subcores=_NS)


def _make_deg_kernel(n, nch):
    nt = n + 8
    npt = n // _NS
    ncs = 125
    nck = npt // ncs
    ect = nch // (_NC * _NS)

    @functools.partial(
        pl.kernel,
        out_type=jax.ShapeDtypeStruct((_NC, n, 16), jnp.float32),
        mesh=_sc_mesh(),
        compiler_params=pltpu.CompilerParams(use_tc_tiling_on_sc=False),
        scratch_types=[
            pltpu.VMEM_SHARED((nt, 16), jnp.float32),
            pltpu.VMEM((_CH, 16), jnp.float32),
            pltpu.VMEM((_CH,), jnp.int32),
            pltpu.VMEM((_CH,), jnp.int32),
            pltpu.SemaphoreType.DMA,
            pltpu.SemaphoreType.DMA,
        ],
    )
    def degk(col_hbm, t2_hbm, t2_sp, onesb, colva, colvb, semsa, semsb):
        c = lax.axis_index("c")
        s = lax.axis_index("s")
        node_base = s * npt
        zero16 = jnp.zeros((16,), jnp.float32)
        one16 = jnp.ones((16,), jnp.float32)

        def zrow(r, carry):
            onesb[r, pl.ds(0, 16)] = zero16
            return carry
        lax.fori_loop(0, _CH, zrow, 0)

        def zcp(ch, carry):
            pltpu.sync_copy(onesb.at[pl.ds(0, ncs)],
                            t2_sp.at[pl.ds(node_base + ch * ncs, ncs)])
            return carry
        lax.fori_loop(0, nck, zcp, 0)

        @pl.when(s == _NS - 1)
        def _():
            pltpu.sync_copy(onesb.at[pl.ds(0, 8)], t2_sp.at[pl.ds(n, 8)])

        def orow(r, carry):
            onesb[r, pl.ds(0, 16)] = one16
            return carry
        lax.fori_loop(0, _CH, orow, 0)

        plsc.subcore_barrier()

        def echunk(jj, carry):
            j = c * (nch // _NC) + s * ect + jj
            pltpu.sync_copy(col_hbm.at[j], colva)
            pltpu.sync_copy(onesb, t2_sp.at[colva], add=True)
            return carry
        lax.fori_loop(0, ect, echunk, 0)

        plsc.subcore_barrier()

        def wout(ch, carry):
            sl = pl.ds(node_base + ch * ncs, ncs)
            pltpu.sync_copy(t2_sp.at[sl], t2_hbm.at[c].at[sl])
            return carry
        lax.fori_loop(0, nck, wout, 0)

    return degk


def _prep_body(t2_ref, x_ref, dinvx_ref, g0_ref, xs_ref):
    t2 = t2_ref[0] + t2_ref[1]
    deg = jnp.sum(t2, axis=1) * (1.0 / 16.0)
    dinv = jnp.where(deg > 0.0, lax.rsqrt(jnp.where(deg > 0.0, deg, 1.0)), 0.0)
    dcol = dinv[:, None]
    blk, d2 = dinvx_ref.shape
    dinvx_ref[...] = jnp.broadcast_to(dcol, (blk, d2))
    x = x_ref[...]
    xs_ref[0] = x[:, :d2]
    xs_ref[1] = x[:, d2:]
    g0_ref[0] = dcol * x[:, :d2]
    g0_ref[1] = dcol * x[:, d2:]


def _make_step_kernel(n, nch):
    nt = n + 8
    npt = n // _NS
    ncs = 125
    nck = npt // ncs
    ect = nch // _NS
    fdim = 64

    vrow = functools.partial(pltpu.VMEM, (ncs, fdim))

    @functools.partial(
        pl.kernel,
        out_type=[jax.ShapeDtypeStruct((_NC * n, fdim), jnp.float32)] * 3,
        mesh=_sc_mesh(),
        compiler_params=pltpu.CompilerParams(use_tc_tiling_on_sc=False),
        scratch_types=[
            pltpu.VMEM_SHARED((nt, fdim), jnp.float32),
            pltpu.VMEM((_CH, fdim), jnp.float32),
            pltpu.VMEM((_CH, fdim), jnp.float32),
            vrow(jnp.float32), vrow(jnp.float32), vrow(jnp.float32),
            vrow(jnp.float32), vrow(jnp.float32), vrow(jnp.float32),
            pltpu.VMEM((ect, _CH), jnp.int32),
            pltpu.VMEM((_CH,), jnp.int32),
            pltpu.VMEM((_CH,), jnp.int32),
            pltpu.VMEM((_CH,), jnp.int32),
            pltpu.VMEM((_CH,), jnp.int32),
            pltpu.VMEM((80,), jnp.float32),
            pltpu.SemaphoreType.DMA,
            pltpu.SemaphoreType.DMA,
            pltpu.SemaphoreType.DMA,
            pltpu.SemaphoreType.DMA,
            pltpu.SemaphoreType.DMA,
            pltpu.SemaphoreType.DMA,
        ],
    )
    def step(pa, pb, g, acc, dinvx, row2, col, coefs,
             pnext, gnext, accout,
             t_sp, eva, evb, tv, pav, pbv, accv, dxv, pnv,
             colb, rowva, rowvb, colva, colvb, cfv,
             semga, semgb, semsa, semsb, semia, semib):
        c = lax.axis_index("c")
        s = lax.axis_index("s")
        node_base = s * npt
        hbm_base = c * n + node_base
        zero16 = jnp.zeros((16,), jnp.float32)

        pltpu.sync_copy(coefs, cfv)

        jt = s * ect
        pltpu.sync_copy(col.at[pl.ds(jt, ect)], colb)
        pltpu.sync_copy(row2.at[c].at[jt], rowva)
        pltpu.sync_copy(row2.at[c].at[jt + 1], rowvb)
        pltpu.async_copy(g.at[rowva], eva, semga)
        pltpu.async_copy(g.at[rowvb], evb, semgb)

        def zrow(r, carry):
            for j2 in range(fdim // 16):
                pnv[r, pl.ds(j2 * 16, 16)] = zero16
            return carry
        lax.fori_loop(0, ncs, zrow, 0)

        def zcp(ch, carry):
            pltpu.sync_copy(pnv, t_sp.at[pl.ds(node_base + ch * ncs, ncs)])
            return carry
        lax.fori_loop(0, nck, zcp, 0)

        @pl.when(s == _NS - 1)
        def _():
            pltpu.sync_copy(pnv.at[pl.ds(0, 8)], t_sp.at[pl.ds(n, 8)])

        plsc.subcore_barrier()

        npair = ect // 2

        def epair(p, carry):
            e = 2 * p

            pltpu.make_async_copy(g.at[rowva], eva, semga).wait()
            pltpu.async_copy(row2.at[c].at[jt + e + 2], rowva, semia)
            for k16 in range(_CH // 16):
                sl = pl.ds(k16 * 16, 16)
                colva[sl] = colb[e, sl]
            da = pltpu.async_copy(eva, t_sp.at[colva], semsa, add=True)

            pltpu.make_async_copy(g.at[rowvb], evb, semgb).wait()
            pltpu.async_copy(row2.at[c].at[jt + e + 3], rowvb, semib)
            for k16 in range(_CH // 16):
                sl = pl.ds(k16 * 16, 16)
                colvb[sl] = colb[e + 1, sl]
            db = pltpu.async_copy(evb, t_sp.at[colvb], semsb, add=True)

            da.wait()
            pltpu.make_async_copy(row2.at[c].at[jt + e + 2], rowva,
                                  semia).wait()
            pltpu.async_copy(g.at[rowva], eva, semga)
            db.wait()
            pltpu.make_async_copy(row2.at[c].at[jt + e + 3], rowvb,
                                  semib).wait()
            pltpu.async_copy(g.at[rowvb], evb, semgb)
            return carry
        lax.fori_loop(0, npair - 1, epair, 0)

        e = ect - 2
        pltpu.make_async_copy(g.at[rowva], eva, semga).wait()
        for k16 in range(_CH // 16):
            sl = pl.ds(k16 * 16, 16)
            colva[sl] = colb[e, sl]
        da = pltpu.async_copy(eva, t_sp.at[colva], semsa, add=True)
        pltpu.make_async_copy(g.at[rowvb], evb, semgb).wait()
        for k16 in range(_CH // 16):
            sl = pl.ds(k16 * 16, 16)
            colvb[sl] = colb[e + 1, sl]
        db = pltpu.async_copy(evb, t_sp.at[colvb], semsb, add=True)
        da.wait()
        db.wait()

        plsc.subcore_barrier()

        bv = cfv[pl.ds(0, 16)]
        cv = cfv[pl.ds(16, 16)]
        av = cfv[pl.ds(32, 16)]
        ov = cfv[pl.ds(48, 16)]
        dv = cfv[pl.ds(64, 16)]

        def nchunk(ch, carry):
            hb = pl.ds(hbm_base + ch * ncs, ncs)
            nb = pl.ds(node_base + ch * ncs, ncs)
            pltpu.sync_copy(t_sp.at[nb], tv)
            pltpu.sync_copy(pa.at[hb], pav)
            pltpu.sync_copy(pb.at[hb], pbv)
            pltpu.sync_copy(acc.at[hb], accv)
            pltpu.sync_copy(dinvx.at[nb], dxv)

            def rrow(r, carry2):
                for j2 in range(fdim // 16):
                    sl = pl.ds(j2 * 16, 16)
                    pb_ = pbv[r, sl]
                    pn = (bv * pb_ + cv * pav[r, sl]
                          + av * (dxv[r, sl] * tv[r, sl]))
                    pnv[r, sl] = pn
                    eva[r, sl] = dxv[r, sl] * pn
                    accv[r, sl] = accv[r, sl] + ov * pn + dv * pb_
                return carry2
            lax.fori_loop(0, ncs, rrow, 0)

            pltpu.sync_copy(pnv, pnext.at[hb])
            pltpu.sync_copy(eva.at[pl.ds(0, ncs)], gnext.at[hb])
            pltpu.sync_copy(accv, accout.at[hb])
            return carry
        lax.fori_loop(0, nck, nchunk, 0)

    return step


def kernel(x, edge_index, lap_coefs, mf_weights):
    n, d = x.shape
    e = edge_index.shape[1]
    row = edge_index[0]
    col = edge_index[1]

    nch = -(-e // _CH)
    nch = -(-nch // (_NC * _NS)) * (_NC * _NS)
    pad = nch * _CH - e
    row_p = jnp.concatenate([row, jnp.zeros((pad,), jnp.int32)])
    col_p = jnp.concatenate([col, jnp.full((pad,), n, jnp.int32)])
    row_r = row_p.reshape(nch, _CH)
    col_r = col_p.reshape(nch, _CH)
    row2 = jnp.stack([row_r, row_r + n])

    t2 = _make_deg_kernel(n, nch)(col_r)
    blk = 1000
    d2 = d // 2
    dinvx, g0, xs = pl.pallas_call(
        _prep_body,
        out_shape=[
            jax.ShapeDtypeStruct((n, d2), jnp.float32),
            jax.ShapeDtypeStruct((_NC, n, d2), jnp.float32),
            jax.ShapeDtypeStruct((_NC, n, d2), jnp.float32),
        ],
        grid=(n // blk,),
        in_specs=[
            pl.BlockSpec((_NC, blk, 16), lambda i: (0, i, 0)),
            pl.BlockSpec((blk, d), lambda i: (i, 0)),
        ],
        out_specs=[
            pl.BlockSpec((blk, d2), lambda i: (i, 0)),
            pl.BlockSpec((_NC, blk, d2), lambda i: (0, i, 0)),
            pl.BlockSpec((_NC, blk, d2), lambda i: (0, i, 0)),
        ],
    )(t2, x)
    g0 = g0.reshape(_NC * n, d2)
    xs = xs.reshape(_NC * n, d2)

    lap = jnp.cumprod(_ALPHA * jnp.tanh(lap_coefs))
    nw = _norm_weights()
    mfw = mf_weights[0, :, 0]
    cout = [mfw[0] / nw[0]]
    for i in range(1, _K + 1):
        cout.append(lap[i - 1] / nw[i] * mfw[i])

    a, b = _adjust_ab(_A, _B)
    c0 = (a - b) / 2.0
    c1 = (a + b + 2.0) / 2.0

    def coef_vec(v):
        return jnp.full((16,), v, jnp.float32)

    step = _make_step_kernel(n, nch)
    pa, pb, g = xs, xs, g0
    acc = jnp.zeros((_NC * n, d2), jnp.float32)
    for kc in range(_K):
        if kc == 0:
            bn, cn, an = c0, 0.0, c1
            dvc = cout[0]
        else:
            an, bn, cn = _jacobi_abc(kc + 1)
            dvc = 0.0
        coefs = jnp.concatenate([
            coef_vec(bn), coef_vec(cn), coef_vec(an),
            coef_vec(cout[kc + 1]), coef_vec(dvc),
        ])
        pnext, gnext, accout = step(pa, pb, g, acc,
                                    dinvx, row2, col_r, coefs)
        pa, pb, g, acc = pb, pnext, gnext, accout

    return jnp.concatenate([acc[:n], acc[n:]], axis=1)
